# baseline (speedup 1.0000x reference)
import functools

import jax
import jax.numpy as jnp
from jax import lax
from jax.experimental import pallas as pl
from jax.experimental.pallas import tpu as pltpu
from jax.experimental.pallas import tpu_sc as plsc

_CAPACITY_FACTOR = 0.125
_MAX_POS = 8192


def _sc_gather(x_flat, flat_idx, n_rows, C):
    info = plsc.get_sparse_core_info()
    nw = info.num_cores * info.num_subcores
    rpw = n_rows // nw
    mesh = plsc.VectorSubcoreMesh(core_axis_name="c", subcore_axis_name="s")

    @functools.partial(
        pl.kernel,
        mesh=mesh,
        out_type=jax.ShapeDtypeStruct((n_rows, C), jnp.float32),
        scratch_types=[
            pltpu.VMEM((rpw,), jnp.int32),
            pltpu.VMEM((rpw, C), jnp.float32),
            pltpu.SemaphoreType.DMA,
        ],
    )
    def gk(x_hbm, idx_hbm, out_hbm, idx_v, rows_v, sem):
        wid = lax.axis_index("s") * info.num_cores + lax.axis_index("c")
        base = wid * rpw
        pltpu.sync_copy(idx_hbm.at[pl.ds(base, rpw)], idx_v)
        pltpu.async_copy(x_hbm.at[idx_v], rows_v, sem).wait()
        pltpu.sync_copy(rows_v, out_hbm.at[pl.ds(base, rpw)])

    return gk(x_flat, flat_idx)


def _sc_scatter_rows(out_ref, rows, flat_idx, n_rows, C):
    info = plsc.get_sparse_core_info()
    nw = info.num_cores * info.num_subcores
    rpw = n_rows // nw
    mesh = plsc.VectorSubcoreMesh(core_axis_name="c", subcore_axis_name="s")

    @functools.partial(
        pl.kernel,
        mesh=mesh,
        scratch_types=[
            pltpu.VMEM((rpw,), jnp.int32),
            pltpu.VMEM((rpw, C), jnp.float32),
            pltpu.SemaphoreType.DMA,
        ],
    )
    def sk(out_hbm, rows_hbm, idx_hbm, idx_v, rows_v, sem):
        wid = lax.axis_index("s") * info.num_cores + lax.axis_index("c")
        base = wid * rpw
        pltpu.sync_copy(idx_hbm.at[pl.ds(base, rpw)], idx_v)
        pltpu.sync_copy(rows_hbm.at[pl.ds(base, rpw)], rows_v)
        pltpu.async_copy(rows_v, out_hbm.at[idx_v], sem).wait()

    sk(out_ref, rows, flat_idx)


def _logits_copy_body(x_ref, wr_ref, cp_ref, lg_ref):
    v = x_ref[...]
    cp_ref[...] = v
    lg_ref[...] = jnp.dot(v, wr_ref[...], preferred_element_type=jnp.float32)


def _mlp_body(w_ref, tok_ref, w1_ref, b1_ref, w2_ref, b2_ref, out_ref):
    t = tok_ref[...]
    h = jax.nn.gelu(jnp.dot(t, w1_ref[...],
                            preferred_element_type=jnp.float32) + b1_ref[...])
    p = t + jnp.dot(h, w2_ref[...],
                    preferred_element_type=jnp.float32) + b2_ref[...]
    out_ref[...] = t + p * w_ref[...]


def kernel(x, position_ids, W_router, W1, b1, W2, b2):
    B, T, C = x.shape
    F = W1.shape[1]
    k = min(int(_CAPACITY_FACTOR * _MAX_POS), int(_CAPACITY_FACTOR * T))
    rows = B * T
    x2 = x.reshape(rows, C)

    lblk = 2048
    xcopy, logits = pl.pallas_call(
        _logits_copy_body,
        grid=(rows // lblk,),
        in_specs=[
            pl.BlockSpec((lblk, C), lambda i: (i, 0)),
            pl.BlockSpec((C, 1), lambda i: (0, 0)),
        ],
        out_specs=[
            pl.BlockSpec((lblk, C), lambda i: (i, 0)),
            pl.BlockSpec((lblk, 1), lambda i: (i, 0)),
        ],
        out_shape=[
            jax.ShapeDtypeStruct((rows, C), jnp.float32),
            jax.ShapeDtypeStruct((rows, 1), jnp.float32),
        ],
    )(x2, W_router)

    weights, sel = jax.lax.top_k(logits.reshape(B, T), k)
    flat_idx = (sel.astype(jnp.int32)
                + (jnp.arange(B, dtype=jnp.int32) * T)[:, None]).reshape(-1)

    gathered = _sc_gather(x2, flat_idx, B * k, C)

    mblk = 512
    final_rows = pl.pallas_call(
        _mlp_body,
        grid=(B * k // mblk,),
        in_specs=[
            pl.BlockSpec((mblk, 1), lambda i: (i, 0)),
            pl.BlockSpec((mblk, C), lambda i: (i, 0)),
            pl.BlockSpec((C, F), lambda i: (0, 0)),
            pl.BlockSpec((1, F), lambda i: (0, 0)),
            pl.BlockSpec((F, C), lambda i: (0, 0)),
            pl.BlockSpec((1, C), lambda i: (0, 0)),
        ],
        out_specs=pl.BlockSpec((mblk, C), lambda i: (i, 0)),
        out_shape=jax.ShapeDtypeStruct((B * k, C), jnp.float32),
    )(weights.reshape(B * k, 1), gathered,
      W1, b1.reshape(1, F), W2, b2.reshape(1, C))

    out_ref = jax.new_ref(xcopy)
    _sc_scatter_rows(out_ref, final_rows, flat_idx, B * k, C)
    return out_ref[...].reshape(B, T, C)


# speedup vs baseline: 1.0131x; 1.0131x over previous
import functools

import jax
import jax.numpy as jnp
from jax import lax
from jax.experimental import pallas as pl
from jax.experimental.pallas import tpu as pltpu
from jax.experimental.pallas import tpu_sc as plsc

_CAPACITY_FACTOR = 0.125
_MAX_POS = 8192


def _sc_gather(x_flat, flat_idx, n_rows, C):
    info = plsc.get_sparse_core_info()
    nw = info.num_cores * info.num_subcores
    rpw = n_rows // nw
    mesh = plsc.VectorSubcoreMesh(core_axis_name="c", subcore_axis_name="s")

    @functools.partial(
        pl.kernel,
        mesh=mesh,
        out_type=jax.ShapeDtypeStruct((n_rows, C), jnp.float32),
        scratch_types=[
            pltpu.VMEM((rpw,), jnp.int32),
            pltpu.VMEM((rpw, C), jnp.float32),
            pltpu.SemaphoreType.DMA,
        ],
    )
    def gk(x_hbm, idx_hbm, out_hbm, idx_v, rows_v, sem):
        wid = lax.axis_index("s") * info.num_cores + lax.axis_index("c")
        base = wid * rpw
        pltpu.sync_copy(idx_hbm.at[pl.ds(base, rpw)], idx_v)
        pltpu.async_copy(x_hbm.at[idx_v], rows_v, sem).wait()
        pltpu.sync_copy(rows_v, out_hbm.at[pl.ds(base, rpw)])

    return gk(x_flat, flat_idx)


def _sc_scatter_rows(out_ref, rows, flat_idx, n_rows, C):
    info = plsc.get_sparse_core_info()
    nw = info.num_cores * info.num_subcores
    rpw = n_rows // nw
    mesh = plsc.VectorSubcoreMesh(core_axis_name="c", subcore_axis_name="s")

    @functools.partial(
        pl.kernel,
        mesh=mesh,
        scratch_types=[
            pltpu.VMEM((rpw,), jnp.int32),
            pltpu.VMEM((rpw, C), jnp.float32),
            pltpu.SemaphoreType.DMA,
        ],
    )
    def sk(out_hbm, rows_hbm, idx_hbm, idx_v, rows_v, sem):
        wid = lax.axis_index("s") * info.num_cores + lax.axis_index("c")
        base = wid * rpw
        pltpu.sync_copy(idx_hbm.at[pl.ds(base, rpw)], idx_v)
        pltpu.sync_copy(rows_hbm.at[pl.ds(base, rpw)], rows_v)
        pltpu.async_copy(rows_v, out_hbm.at[idx_v], sem).wait()

    sk(out_ref, rows, flat_idx)


def _logits_copy_body(x_ref, wr_ref, cp_ref, lg_ref):
    v = x_ref[...]
    cp_ref[...] = v
    lg_ref[...] = jnp.dot(v, wr_ref[...], preferred_element_type=jnp.float32)


def _mlp_body(w_ref, tok_ref, w1_ref, b1_ref, w2_ref, b2_ref, out_ref):
    t = tok_ref[...]
    h = jax.nn.gelu(jnp.dot(t, w1_ref[...],
                            preferred_element_type=jnp.float32) + b1_ref[...])
    p = t + jnp.dot(h, w2_ref[...],
                    preferred_element_type=jnp.float32) + b2_ref[...]
    out_ref[...] = t + p * w_ref[...]


def kernel(x, position_ids, W_router, W1, b1, W2, b2):
    B, T, C = x.shape
    F = W1.shape[1]
    k = min(int(_CAPACITY_FACTOR * _MAX_POS), int(_CAPACITY_FACTOR * T))
    rows = B * T
    x2 = x.reshape(rows, C)

    lblk = 4096
    xcopy, logits = pl.pallas_call(
        _logits_copy_body,
        grid=(rows // lblk,),
        in_specs=[
            pl.BlockSpec((lblk, C), lambda i: (i, 0)),
            pl.BlockSpec((C, 1), lambda i: (0, 0)),
        ],
        out_specs=[
            pl.BlockSpec((lblk, C), lambda i: (i, 0)),
            pl.BlockSpec((lblk, 1), lambda i: (i, 0)),
        ],
        out_shape=[
            jax.ShapeDtypeStruct((rows, C), jnp.float32),
            jax.ShapeDtypeStruct((rows, 1), jnp.float32),
        ],
    )(x2, W_router)

    weights, sel = jax.lax.top_k(logits.reshape(B, T), k)
    flat_idx = (sel.astype(jnp.int32)
                + (jnp.arange(B, dtype=jnp.int32) * T)[:, None]).reshape(-1)

    gathered = _sc_gather(x2, flat_idx, B * k, C)

    mblk = 512
    final_rows = pl.pallas_call(
        _mlp_body,
        grid=(B * k // mblk,),
        in_specs=[
            pl.BlockSpec((mblk, 1), lambda i: (i, 0)),
            pl.BlockSpec((mblk, C), lambda i: (i, 0)),
            pl.BlockSpec((C, F), lambda i: (0, 0)),
            pl.BlockSpec((1, F), lambda i: (0, 0)),
            pl.BlockSpec((F, C), lambda i: (0, 0)),
            pl.BlockSpec((1, C), lambda i: (0, 0)),
        ],
        out_specs=pl.BlockSpec((mblk, C), lambda i: (i, 0)),
        out_shape=jax.ShapeDtypeStruct((B * k, C), jnp.float32),
    )(weights.reshape(B * k, 1), gathered,
      W1, b1.reshape(1, F), W2, b2.reshape(1, C))

    out_ref = jax.new_ref(xcopy)
    _sc_scatter_rows(out_ref, final_rows, flat_idx, B * k, C)
    return out_ref[...].reshape(B, T, C)


# speedup vs baseline: 1.0170x; 1.0039x over previous
import functools

import jax
import jax.numpy as jnp
from jax import lax
from jax.experimental import pallas as pl
from jax.experimental.pallas import tpu as pltpu
from jax.experimental.pallas import tpu_sc as plsc

_CAPACITY_FACTOR = 0.125
_MAX_POS = 8192


def _sc_gather(x_flat, flat_idx, n_rows, C):
    info = plsc.get_sparse_core_info()
    nw = info.num_cores * info.num_subcores
    rpw = n_rows // nw
    mesh = plsc.VectorSubcoreMesh(core_axis_name="c", subcore_axis_name="s")

    @functools.partial(
        pl.kernel,
        mesh=mesh,
        out_type=jax.ShapeDtypeStruct((n_rows, C), jnp.float32),
        scratch_types=[
            pltpu.VMEM((rpw,), jnp.int32),
            pltpu.VMEM((rpw, C), jnp.float32),
            pltpu.SemaphoreType.DMA,
        ],
    )
    def gk(x_hbm, idx_hbm, out_hbm, idx_v, rows_v, sem):
        wid = lax.axis_index("s") * info.num_cores + lax.axis_index("c")
        base = wid * rpw
        pltpu.sync_copy(idx_hbm.at[pl.ds(base, rpw)], idx_v)
        pltpu.async_copy(x_hbm.at[idx_v], rows_v, sem).wait()
        pltpu.sync_copy(rows_v, out_hbm.at[pl.ds(base, rpw)])

    return gk(x_flat, flat_idx)


def _sc_scatter_rows(out_ref, rows, flat_idx, n_rows, C):
    info = plsc.get_sparse_core_info()
    nw = info.num_cores * info.num_subcores
    rpw = n_rows // nw
    mesh = plsc.VectorSubcoreMesh(core_axis_name="c", subcore_axis_name="s")

    @functools.partial(
        pl.kernel,
        mesh=mesh,
        scratch_types=[
            pltpu.VMEM((rpw,), jnp.int32),
            pltpu.VMEM((rpw, C), jnp.float32),
            pltpu.SemaphoreType.DMA,
        ],
    )
    def sk(out_hbm, rows_hbm, idx_hbm, idx_v, rows_v, sem):
        wid = lax.axis_index("s") * info.num_cores + lax.axis_index("c")
        base = wid * rpw
        pltpu.sync_copy(idx_hbm.at[pl.ds(base, rpw)], idx_v)
        pltpu.sync_copy(rows_hbm.at[pl.ds(base, rpw)], rows_v)
        pltpu.async_copy(rows_v, out_hbm.at[idx_v], sem).wait()

    sk(out_ref, rows, flat_idx)


def _logits_copy_body(x_ref, wr_ref, cp_ref, lg_ref):
    v = x_ref[...]
    cp_ref[...] = v
    lg_ref[...] = jnp.dot(v, wr_ref[...], preferred_element_type=jnp.float32)


def _mlp_body(w_ref, tok_ref, w1_ref, b1_ref, w2_ref, b2_ref, out_ref):
    t = tok_ref[...]
    h = jax.nn.gelu(jnp.dot(t, w1_ref[...],
                            preferred_element_type=jnp.float32) + b1_ref[...])
    p = t + jnp.dot(h, w2_ref[...],
                    preferred_element_type=jnp.float32) + b2_ref[...]
    out_ref[...] = t + p * w_ref[...]


def kernel(x, position_ids, W_router, W1, b1, W2, b2):
    B, T, C = x.shape
    F = W1.shape[1]
    k = min(int(_CAPACITY_FACTOR * _MAX_POS), int(_CAPACITY_FACTOR * T))
    rows = B * T
    x2 = x.reshape(rows, C)

    lblk = 4096
    xcopy, logits = pl.pallas_call(
        _logits_copy_body,
        grid=(rows // lblk,),
        in_specs=[
            pl.BlockSpec((lblk, C), lambda i: (i, 0)),
            pl.BlockSpec((C, 1), lambda i: (0, 0)),
        ],
        out_specs=[
            pl.BlockSpec((lblk, C), lambda i: (i, 0)),
            pl.BlockSpec((lblk, 1), lambda i: (i, 0)),
        ],
        out_shape=[
            jax.ShapeDtypeStruct((rows, C), jnp.float32),
            jax.ShapeDtypeStruct((rows, 1), jnp.float32),
        ],
    )(x2, W_router)

    weights, sel = jax.lax.top_k(logits.reshape(B, T), k)
    flat_idx = (sel.astype(jnp.int32)
                + (jnp.arange(B, dtype=jnp.int32) * T)[:, None]).reshape(-1)

    gathered = _sc_gather(x2, flat_idx, B * k, C)

    mblk = 1024
    final_rows = pl.pallas_call(
        _mlp_body,
        grid=(B * k // mblk,),
        in_specs=[
            pl.BlockSpec((mblk, 1), lambda i: (i, 0)),
            pl.BlockSpec((mblk, C), lambda i: (i, 0)),
            pl.BlockSpec((C, F), lambda i: (0, 0)),
            pl.BlockSpec((1, F), lambda i: (0, 0)),
            pl.BlockSpec((F, C), lambda i: (0, 0)),
            pl.BlockSpec((1, C), lambda i: (0, 0)),
        ],
        out_specs=pl.BlockSpec((mblk, C), lambda i: (i, 0)),
        out_shape=jax.ShapeDtypeStruct((B * k, C), jnp.float32),
    )(weights.reshape(B * k, 1), gathered,
      W1, b1.reshape(1, F), W2, b2.reshape(1, C))

    out_ref = jax.new_ref(xcopy)
    _sc_scatter_rows(out_ref, final_rows, flat_idx, B * k, C)
    return out_ref[...].reshape(B, T, C)
